# baseline (speedup 1.0000x reference)
import jax
import jax.numpy as jnp
from jax import lax
from jax.experimental import pallas as pl
from jax.experimental.pallas import tpu as pltpu
from jax.experimental.pallas import tpu_sc as plsc

NC = 2
NS = 16
NW = NC * NS

TABLE_ROWS = 256 * 256 * 256

N_ELEMS = 4 * 3 * 512 * 512
N_TRIPLES = N_ELEMS // 3
TRIPLES_PER_TILE = N_TRIPLES // NW
C = 4096
E = 3 * C
N_CHUNKS = TRIPLES_PER_TILE // C
ELEMS_PER_TILE = N_ELEMS // NW


def _sc_body(x_hbm, w0, w1, w2, k0, k1, k2, out_hbm,
             xv0, xv1, ix0, ix1, wv0, wv1, kv0, kv1, outv,
             sx0, sx1, sw0, sw1, sk0, sk1):
    wid = lax.axis_index("s") * NC + lax.axis_index("c")
    base0 = wid * ELEMS_PER_TILE
    iota = lax.iota(jnp.int32, 16)
    xv = (xv0, xv1)
    ix = (ix0, ix1)
    wv = (wv0, wv1)
    kv = (kv0, kv1)
    sx = (sx0, sx1)
    sw = (sw0, sw1)
    sk = (sk0, sk1)

    def fire_x(n):
        e0 = base0 + n * E
        return pltpu.async_copy(x_hbm.at[pl.ds(e0, E)], xv[n % 2], sx[n % 2])

    def build_and_fire(n):
        b = n % 2

        def idx_body(it, _):
            p = it * 48 + iota * 3
            r = plsc.load_gather(xv[b], [p])
            g = plsc.load_gather(xv[b], [p + 1])
            bb = plsc.load_gather(xv[b], [p + 2])
            ind = (r * 65536.0 + g * 256.0 + bb).astype(jnp.int32)
            plsc.store_scatter(ix[b], [it * 16 + iota], ind)
            return 0

        lax.fori_loop(0, C // 16, idx_body, 0)
        cps = []
        for j, t in enumerate((w0, w1, w2)):
            cps.append(pltpu.async_copy(
                t.at[ix[b]], wv[b].at[pl.ds(j * C, C)], sw[b]))
        for j, t in enumerate((k0, k1, k2)):
            cps.append(pltpu.async_copy(
                t.at[ix[b]], kv[b].at[pl.ds(j * C, C)], sk[b]))
        return cps

    def drain(n, cps):
        b = n % 2
        for cp in cps:
            cp.wait()

        def ew_body(u, _):
            sl = pl.ds(u * 16, 16)
            f = u * 16 + iota
            t3 = lax.shift_right_logical(f * 21846, 16)
            j3 = f - t3 * 3
            a = j3 * C + t3
            wz = plsc.load_gather(wv[b], [a])
            kz = plsc.load_gather(kv[b], [a])
            z = xv[b][sl] * wz + kz
            t = jnp.exp(-2.0 * jnp.abs(z))
            outv[sl] = jnp.sign(z) * ((1.0 - t) / (1.0 + t))
            return 0

        lax.fori_loop(0, E // 16, ew_body, 0)
        e0 = base0 + n * E
        pltpu.sync_copy(outv, out_hbm.at[pl.ds(e0, E)])

    fire_x(0).wait()
    pending = build_and_fire(0)
    nxt = fire_x(1)
    for n in range(1, N_CHUNKS):
        nxt.wait()
        cps = build_and_fire(n)
        drain(n - 1, pending)
        if n + 1 < N_CHUNKS:
            nxt = fire_x(n + 1)
        pending = cps
    drain(N_CHUNKS - 1, pending)


@jax.jit
def _sc_call(xf, w0, w1, w2, k0, k1, k2):
    mesh = plsc.VectorSubcoreMesh(
        core_axis_name="c", subcore_axis_name="s",
        num_cores=NC, num_subcores=NS)
    f = pl.kernel(
        _sc_body,
        out_type=jax.ShapeDtypeStruct((N_ELEMS,), jnp.float32),
        mesh=mesh,
        scratch_types=(
            [pltpu.VMEM((E,), jnp.float32)] * 2
            + [pltpu.VMEM((C,), jnp.int32)] * 2
            + [pltpu.VMEM((E,), jnp.float32)] * 2
            + [pltpu.VMEM((E,), jnp.float32)] * 2
            + [pltpu.VMEM((E,), jnp.float32)]
            + [pltpu.SemaphoreType.DMA] * 6
        ),
        compiler_params=pltpu.CompilerParams(needs_layout_passes=False),
    )
    return f(xf, w0, w1, w2, k0, k1, k2)


def kernel(x, w, k):
    b, c, h, wd = x.shape
    out = _sc_call(x.reshape(-1),
                   w[:, 0], w[:, 1], w[:, 2],
                   k[:, 0], k[:, 1], k[:, 2])
    return out.reshape(-1, 3, h, wd)


# speedup vs baseline: 1.0207x; 1.0207x over previous
import jax
import jax.numpy as jnp
from jax import lax
from jax.experimental import pallas as pl
from jax.experimental.pallas import tpu as pltpu
from jax.experimental.pallas import tpu_sc as plsc

NC = 2
NS = 16
NW = NC * NS

TABLE_ROWS = 256 * 256 * 256

N_ELEMS = 4 * 3 * 512 * 512
N_TRIPLES = N_ELEMS // 3
TRIPLES_PER_TILE = N_TRIPLES // NW
C = 2048
E = 3 * C
N_CHUNKS = TRIPLES_PER_TILE // C
ELEMS_PER_TILE = N_ELEMS // NW


def _sc_body(x_hbm, w0, w1, w2, k0, k1, k2, out_hbm,
             xv0, xv1, ix0, ix1, wv0, wv1, kv0, kv1, outv,
             sx0, sx1, sw0, sw1, sk0, sk1):
    wid = lax.axis_index("s") * NC + lax.axis_index("c")
    base0 = wid * ELEMS_PER_TILE
    iota = lax.iota(jnp.int32, 16)
    xv = (xv0, xv1)
    ix = (ix0, ix1)
    wv = (wv0, wv1)
    kv = (kv0, kv1)
    sx = (sx0, sx1)
    sw = (sw0, sw1)
    sk = (sk0, sk1)

    def fire_x(n):
        e0 = base0 + n * E
        return pltpu.async_copy(x_hbm.at[pl.ds(e0, E)], xv[n % 2], sx[n % 2])

    def build_and_fire(n):
        b = n % 2

        def idx_body(it, _):
            p = it * 48 + iota * 3
            r = plsc.load_gather(xv[b], [p])
            g = plsc.load_gather(xv[b], [p + 1])
            bb = plsc.load_gather(xv[b], [p + 2])
            ind = (r * 65536.0 + g * 256.0 + bb).astype(jnp.int32)
            plsc.store_scatter(ix[b], [it * 16 + iota], ind)
            return 0

        lax.fori_loop(0, C // 16, idx_body, 0)
        cps = []
        for j, t in enumerate((w0, w1, w2)):
            cps.append(pltpu.async_copy(
                t.at[ix[b]], wv[b].at[pl.ds(j * C, C)], sw[b]))
        for j, t in enumerate((k0, k1, k2)):
            cps.append(pltpu.async_copy(
                t.at[ix[b]], kv[b].at[pl.ds(j * C, C)], sk[b]))
        return cps

    def drain(n, cps):
        b = n % 2
        for cp in cps:
            cp.wait()

        def ew_body(u, _):
            sl = pl.ds(u * 16, 16)
            f = u * 16 + iota
            t3 = lax.shift_right_logical(f * 21846, 16)
            j3 = f - t3 * 3
            a = j3 * C + t3
            wz = plsc.load_gather(wv[b], [a])
            kz = plsc.load_gather(kv[b], [a])
            z = xv[b][sl] * wz + kz
            t = jnp.exp(-2.0 * jnp.abs(z))
            outv[sl] = jnp.sign(z) * ((1.0 - t) / (1.0 + t))
            return 0

        lax.fori_loop(0, E // 16, ew_body, 0)
        e0 = base0 + n * E
        pltpu.sync_copy(outv, out_hbm.at[pl.ds(e0, E)])

    fire_x(0).wait()
    pending = build_and_fire(0)
    nxt = fire_x(1)
    for n in range(1, N_CHUNKS):
        nxt.wait()
        cps = build_and_fire(n)
        drain(n - 1, pending)
        if n + 1 < N_CHUNKS:
            nxt = fire_x(n + 1)
        pending = cps
    drain(N_CHUNKS - 1, pending)


@jax.jit
def _sc_call(xf, w0, w1, w2, k0, k1, k2):
    mesh = plsc.VectorSubcoreMesh(
        core_axis_name="c", subcore_axis_name="s",
        num_cores=NC, num_subcores=NS)
    f = pl.kernel(
        _sc_body,
        out_type=jax.ShapeDtypeStruct((N_ELEMS,), jnp.float32),
        mesh=mesh,
        scratch_types=(
            [pltpu.VMEM((E,), jnp.float32)] * 2
            + [pltpu.VMEM((C,), jnp.int32)] * 2
            + [pltpu.VMEM((E,), jnp.float32)] * 2
            + [pltpu.VMEM((E,), jnp.float32)] * 2
            + [pltpu.VMEM((E,), jnp.float32)]
            + [pltpu.SemaphoreType.DMA] * 6
        ),
        compiler_params=pltpu.CompilerParams(needs_layout_passes=False),
    )
    return f(xf, w0, w1, w2, k0, k1, k2)


def kernel(x, w, k):
    b, c, h, wd = x.shape
    out = _sc_call(x.reshape(-1),
                   w[:, 0], w[:, 1], w[:, 2],
                   k[:, 0], k[:, 1], k[:, 2])
    return out.reshape(-1, 3, h, wd)
